# baseline (speedup 1.0000x reference)
import functools

import jax
import jax.numpy as jnp
from jax import lax
from jax.experimental import pallas as pl
from jax.experimental.pallas import tpu as pltpu
from jax.experimental.pallas import tpu_sc as plsc

_NUM_CLASSES = 80
_ATTRS = _NUM_CLASSES + 5
_INPUT_SIZE = 416.0
_ANCHORS = ((10., 13.), (16., 30.), (33., 23.), (30., 61.), (62., 45.),
            (59., 119.), (116., 90.), (156., 198.), (373., 326.))
_MASK = ((6, 7, 8), (3, 4, 5), (0, 1, 2))
_BALANCE = (0.4, 1.0, 4.0)
_BOX_RATIO = 0.05
_OBJ_RATIO = 5.0
_CLS_RATIO = _NUM_CLASSES / 80.0
_EPS = 1e-7
_B = 16
_M = 32
_A = 3
_NBOX = _B * _M
_NSC = 32
_BPW = _NBOX // _NSC
_ROWS = _BPW * _ATTRS
_RPAD = 1408
_HWS = ((13, 13), (26, 26), (52, 52))


def _softplus(x):
    return jnp.maximum(x, 0.0) + jnp.log(1.0 + jnp.exp(-jnp.abs(x)))


def _box_prologue(bxv, byv, bwv, bhv, lt):
    H, W = _HWS[lt]
    stride = _INPUT_SIZE / H
    sa = tuple((aw / stride, ah / stride) for aw, ah in _ANCHORS)
    mids = _MASK[lt]
    Hf, Wf = float(H), float(W)
    gx = bxv * Wf
    gy = byv * Hf
    gw = jnp.maximum(bwv * Wf, 1e-6)
    gh = jnp.maximum(bhv * Hf, 1e-6)
    valid = (bwv > 1e-6) & (bhv > 1e-6)

    def anch_iou(aw, ah):
        inter = jnp.minimum(gw, aw) * jnp.minimum(gh, ah)
        union = gw * gh + aw * ah - inter
        return inter / jnp.maximum(union, 1e-9)

    b_iou = anch_iou(*sa[0])
    b_idx = jnp.zeros(gx.shape, jnp.int32)
    b_aw = jnp.full(gx.shape, sa[0][0], jnp.float32)
    b_ah = jnp.full(gx.shape, sa[0][1], jnp.float32)
    for i in range(1, 9):
        iou_i = anch_iou(*sa[i])
        upd = iou_i > b_iou
        b_iou = jnp.where(upd, iou_i, b_iou)
        b_idx = jnp.where(upd, i, b_idx)
        b_aw = jnp.where(upd, sa[i][0], b_aw)
        b_ah = jnp.where(upd, sa[i][1], b_ah)

    in_layer = ((b_idx == mids[0]) | (b_idx == mids[1]) | (b_idx == mids[2]))
    a_sel = jnp.where(b_idx == mids[0], 0,
                      jnp.where(b_idx == mids[1], 1, 2)).astype(jnp.int32)
    gi = jnp.clip(jnp.floor(gx).astype(jnp.int32), 0, W - 1)
    gj = jnp.clip(jnp.floor(gy).astype(jnp.int32), 0, H - 1)
    ok = valid & in_layer
    return dict(gx=gx, gy=gy, gw=gw, gh=gh, valid=valid, ok=ok,
                a_sel=a_sel, gi=gi, gj=gj, b_aw=b_aw, b_ah=b_ah,
                H=H, W=W)


def _prologue_body(bx_ref, by_ref, bw_ref, bh_ref, base_ref, win_ref):
    bxv = bx_ref[...]
    byv = by_ref[...]
    bwv = bw_ref[...]
    bhv = bh_ref[...]
    bidx = lax.broadcasted_iota(jnp.int32, (_B, _M), 0)
    mi = lax.broadcasted_iota(jnp.int32, (_B, _M, _M), 1)
    mj = lax.broadcasted_iota(jnp.int32, (_B, _M, _M), 2)
    upper = mj > mi
    for lt in range(_A):
        p = _box_prologue(bxv, byv, bwv, bhv, lt)
        H, W = p["H"], p["W"]
        HW = H * W
        ok, a_sel, gi, gj = p["ok"], p["a_sel"], p["gi"], p["gj"]
        key = a_sel * HW + gj * W + gi
        conflict = jnp.any(ok[:, None, :] & (key[:, :, None] == key[:, None, :])
                           & upper, axis=2)
        win = ok & jnp.logical_not(conflict)
        base = (bidx * (_A * _ATTRS) + a_sel * _ATTRS) * HW + gj * W + gi
        base_ref[lt] = jnp.where(win, base, 0)
        win_ref[lt] = win.astype(jnp.int32)


def _run_prologue(bx, by, bw, bh):
    spec = pl.BlockSpec((_B, _M), lambda _: (0, 0))
    out = pl.pallas_call(
        _prologue_body,
        grid=(1,),
        in_specs=[spec, spec, spec, spec],
        out_specs=(pl.BlockSpec((_A, _B, _M), lambda _: (0, 0, 0)),
                   pl.BlockSpec((_A, _B, _M), lambda _: (0, 0, 0))),
        out_shape=(jax.ShapeDtypeStruct((_A, _B, _M), jnp.int32),
                   jax.ShapeDtypeStruct((_A, _B, _M), jnp.int32)),
    )(bx, by, bw, bh)
    return out


_NCH = -(-_ROWS // 128)


_NBUF = 4


def _sc_body(xd_ref, xm_ref, xs_ref, bases_ref, out_ref,
             row_v, lane_v, out_v, buf0, buf1, buf2, buf3, base_v, sem):
    nc = 2
    wid = lax.axis_index("s") * nc + lax.axis_index("c")
    tables = (xd_ref, xm_ref, xs_ref)
    bufs = (buf0, buf1, buf2, buf3)
    lane16 = lax.broadcasted_iota(jnp.int32, (_BPW,), 0)
    for l in range(_A):
        H, W = _HWS[l]
        HW = H * W
        pltpu.sync_copy(bases_ref.at[l, 0, pl.ds(wid * _BPW, _BPW)], base_v)
        bvec = base_v[...]

        def build(c, carry):
            idx = bvec + c * HW
            row_v[pl.ds(c * _BPW, _BPW)] = jnp.right_shift(idx, 7)
            lane_v[pl.ds(c * _BPW, _BPW)] = jnp.bitwise_and(idx, 127)
            return carry

        lax.fori_loop(0, _ATTRS, build, 0)

        def fire(t):
            sz = min(128, _ROWS - t * 128)
            dst = bufs[t % _NBUF]
            if sz < 128:
                dst = dst.at[pl.ds(0, sz), :]
            return pltpu.async_copy(
                tables[l].at[row_v.at[pl.ds(t * 128, sz)]], dst, sem)

        cps = [fire(t) for t in range(_NBUF - 1)]
        for t in range(_NCH):
            if t + _NBUF - 1 < _NCH:
                cps.append(fire(t + _NBUF - 1))
            cps[t].wait()
            buf = bufs[t % _NBUF]
            for u in range(min(128, _ROWS - t * 128) // _BPW):
                p = t * 128 + u * _BPW
                vals = plsc.load_gather(
                    buf, [u * _BPW + lane16, lane_v[pl.ds(p, _BPW)]])
                out_v[pl.ds(p, _BPW)] = vals
        pltpu.sync_copy(out_v, out_ref.at[l, 0, pl.ds(wid * _RPAD, _RPAD)])


def _sc_gather(xd, xm, xs, bases):
    mesh = plsc.VectorSubcoreMesh(core_axis_name="c", subcore_axis_name="s")
    f = pl.kernel(
        _sc_body,
        out_type=jax.ShapeDtypeStruct((_A, 1, _NSC * _RPAD), jnp.float32),
        mesh=mesh,
        compiler_params=pltpu.CompilerParams(needs_layout_passes=False),
        scratch_types=[
            pltpu.VMEM((_ROWS,), jnp.int32),
            pltpu.VMEM((_ROWS,), jnp.int32),
            pltpu.VMEM((_RPAD,), jnp.float32),
            pltpu.VMEM((128, 128), jnp.float32),
            pltpu.VMEM((128, 128), jnp.float32),
            pltpu.VMEM((128, 128), jnp.float32),
            pltpu.VMEM((128, 128), jnp.float32),
            pltpu.VMEM((_BPW,), jnp.int32),
            pltpu.SemaphoreType.DMA,
        ],
    )
    return f(xd, xm, xs, bases)


def _pad_copy_body(x_ref, o_ref):
    o_ref[...] = x_ref[...]


def _as_rows128(pred):
    f = pred.reshape(-1)
    n = f.shape[0]
    npad = -(-n // 128) * 128
    if npad == n:
        return f.reshape(-1, 128)
    blk = 128 * 1024
    grid = -(-npad // blk)
    out = pl.pallas_call(
        _pad_copy_body,
        grid=(grid,),
        in_specs=[pl.BlockSpec((blk,), lambda i: (i,))],
        out_specs=pl.BlockSpec((blk,), lambda i: (i,)),
        out_shape=jax.ShapeDtypeStruct((npad,), jnp.float32),
    )(f)
    return out.reshape(-1, 128)


def _dense_body(lt, bx_ref, by_ref, bw_ref, bh_ref,
                x0_ref, x1_ref, x2_ref, out_ref):
    H, W = _HWS[lt]
    HW = H * W
    stride = _INPUT_SIZE / H
    sa = tuple((aw / stride, ah / stride) for aw, ah in _ANCHORS)
    mids = _MASK[lt]
    Hf, Wf = float(H), float(W)
    xrefs = (x0_ref, x1_ref, x2_ref)

    p = _box_prologue(bx_ref[...], by_ref[...], bw_ref[...], bh_ref[...], lt)
    gx, gy, gw, gh, valid = p["gx"], p["gy"], p["gw"], p["gh"], p["valid"]
    g1x = gx - gw / 2
    g2x = gx + gw / 2
    g1y = gy - gh / 2
    g2y = gy + gh / 2
    area_g = (g2x - g1x) * (g2y - g1y)

    q = lax.broadcasted_iota(jnp.int32, (_B, HW), 1)
    jj = q // W
    ii = q - jj * W
    iif = ii.astype(jnp.float32)
    jjf = jj.astype(jnp.float32)

    s4 = 0.0
    for a in range(_A):
        X = xrefs[a]
        xp = X[:, 0, 0, :]
        yp = X[:, 0, 1, :]
        pw = X[:, 0, 2, :]
        ph = X[:, 0, 3, :]
        xc = X[:, 0, 4, :]
        law, lah = sa[mids[a]]
        bxg = jax.nn.sigmoid(xp) + iif
        byg = jax.nn.sigmoid(yp) + jjf
        bwg = jnp.exp(pw) * law
        bhg = jnp.exp(ph) * lah
        p1x = bxg - bwg / 2
        p2x = bxg + bwg / 2
        p1y = byg - bhg / 2
        p2y = byg + bhg / 2
        area_p = (p2x - p1x) * (p2y - p1y)

        ign = jnp.zeros((_B, HW), jnp.bool_)
        for m in range(_M):
            sl = lambda arr: arr[:, m:m + 1]
            iw = jnp.maximum(jnp.minimum(p2x, sl(g2x)) - jnp.maximum(p1x, sl(g1x)), 0.0)
            ih = jnp.maximum(jnp.minimum(p2y, sl(g2y)) - jnp.maximum(p1y, sl(g1y)), 0.0)
            inter = iw * ih
            den = jnp.maximum(area_p + sl(area_g) - inter, 1e-9)
            ign = ign | (sl(valid) & (inter + inter > den))

        s4 = s4 + jnp.sum(jnp.logical_not(ign).astype(jnp.float32) * _softplus(xc))

    out_ref[0] = s4


def _run_dense(pred, bx, by, bw, bh, lt):
    B, C, H, W = pred.shape
    HW = H * W
    x = pred.reshape(B, C // 5, 5, HW)
    vec_spec = pl.BlockSpec((_B, _M), lambda _: (0, 0))

    def xspec(a):
        return pl.BlockSpec((B, 1, 5, HW), lambda _, a=a: (0, (_ATTRS // 5) * a, 0, 0))

    return pl.pallas_call(
        functools.partial(_dense_body, lt),
        grid=(1,),
        in_specs=[vec_spec, vec_spec, vec_spec, vec_spec,
                  xspec(0), xspec(1), xspec(2)],
        out_specs=pl.BlockSpec(memory_space=pltpu.SMEM),
        out_shape=jax.ShapeDtypeStruct((8,), jnp.float32),
    )(bx, by, bw, bh, x, x, x)


def _epilogue_body(g_ref, lab_ref, win_ref, bx_ref, by_ref, bw_ref, bh_ref,
                   bxr_ref, byr_ref, bwr_ref, bhr_ref, out_ref):
    lab = lab_ref[...]
    cid = lax.broadcasted_iota(jnp.int32, (_NSC, _NUM_CLASSES, _BPW), 1)
    tt = (lab[:, None, :] == cid).astype(jnp.float32)
    bxv, byv, bwv, bhv = bx_ref[...], by_ref[...], bw_ref[...], bh_ref[...]
    bxr, byr, bwr, bhr = bxr_ref[...], byr_ref[...], bwr_ref[...], bhr_ref[...]
    for l in range(_A):
        H, W = _HWS[l]
        stride = _INPUT_SIZE / H
        sa = tuple((aw / stride, ah / stride) for aw, ah in _ANCHORS)
        mids = _MASK[l]
        Hf, Wf = float(H), float(W)

        p = _box_prologue(bxv, byv, bwv, bhv, l)
        gx, gy, gw, gh = p["gx"], p["gy"], p["gw"], p["gh"]
        a_sel, gi, gj = p["a_sel"], p["gi"], p["gj"]
        b_aw, b_ah = p["b_aw"], p["b_ah"]
        gif = gi.astype(jnp.float32)
        gjf = gj.astype(jnp.float32)
        tx = gx - gif
        ty = gy - gjf
        tw = jnp.log(gw / b_aw)
        th = jnp.log(gh / b_ah)
        scl = gw * gh / (Hf * Wf)

        wf = (win_ref[l] > 0).astype(jnp.float32)
        X = g_ref[l]
        xp = X[:, 0, :]
        yp = X[:, 1, :]
        pw = X[:, 2, :]
        ph = X[:, 3, :]
        xc = X[:, 4, :]
        cls = X[:, 5:5 + _NUM_CLASSES, :]

        bsc = 2.0 - scl
        s1 = jnp.sum(wf * bsc * (_softplus(xp) - tx * xp + _softplus(yp) - ty * yp))
        s2 = jnp.sum(wf * bsc * 0.5 * ((pw - tw) ** 2 + (ph - th) ** 2))
        spc = _softplus(xc)
        s3 = jnp.sum(wf * (spc - xc))

        law = jnp.where(a_sel == 0, sa[mids[0]][0],
                        jnp.where(a_sel == 1, sa[mids[1]][0], sa[mids[2]][0]))
        lah = jnp.where(a_sel == 0, sa[mids[0]][1],
                        jnp.where(a_sel == 1, sa[mids[1]][1], sa[mids[2]][1]))
        bxg = jax.nn.sigmoid(xp) + gif
        byg = jax.nn.sigmoid(yp) + gjf
        bwg = jnp.exp(pw) * law
        bhg = jnp.exp(ph) * lah
        p1x = bxg - bwg / 2
        p2x = bxg + bwg / 2
        p1y = byg - bhg / 2
        p2y = byg + bhg / 2
        area_p = (p2x - p1x) * (p2y - p1y)

        gxr = bxr * Wf
        gyr = byr * Hf
        gwr = jnp.maximum(bwr * Wf, 1e-6)
        ghr = jnp.maximum(bhr * Hf, 1e-6)
        validr = (bwr > 1e-6) & (bhr > 1e-6)
        g1xr = gxr - gwr / 2
        g2xr = gxr + gwr / 2
        g1yr = gyr - ghr / 2
        g2yr = gyr + ghr / 2
        area_gr = (g2xr - g1xr) * (g2yr - g1yr)

        ign = jnp.zeros((_NSC, _BPW), jnp.bool_)
        for m in range(_M):
            sl = lambda arr: arr[:, m:m + 1]
            iw = jnp.maximum(jnp.minimum(p2x, sl(g2xr)) - jnp.maximum(p1x, sl(g1xr)), 0.0)
            ih = jnp.maximum(jnp.minimum(p2y, sl(g2yr)) - jnp.maximum(p1y, sl(g1yr)), 0.0)
            inter = iw * ih
            den = jnp.maximum(area_p + sl(area_gr) - inter, 1e-9)
            ign = ign | (sl(validr) & (inter + inter > den))

        s4c = jnp.sum(wf * jnp.logical_not(ign).astype(jnp.float32) * spc)
        s5 = jnp.sum(wf[:, None, :] * (_softplus(cls) - tt * cls))
        npos = jnp.sum(wf)

        out_ref[l, 0] = s1
        out_ref[l, 1] = s2
        out_ref[l, 2] = s3
        out_ref[l, 3] = s4c
        out_ref[l, 4] = s5
        out_ref[l, 5] = npos


def _run_epilogue(gath, labels2, win2, box2, boxr):
    s2d = pl.BlockSpec((_NSC, _BPW), lambda _: (0, 0))
    srep = pl.BlockSpec((_NSC, _M), lambda _: (0, 0))
    return pl.pallas_call(
        _epilogue_body,
        grid=(1,),
        in_specs=[pl.BlockSpec((_A, _NSC, _RPAD // _BPW, _BPW),
                               lambda _: (0, 0, 0, 0)),
                  s2d,
                  pl.BlockSpec((_A, _NSC, _BPW), lambda _: (0, 0, 0)),
                  s2d, s2d, s2d, s2d,
                  srep, srep, srep, srep],
        out_specs=pl.BlockSpec(memory_space=pltpu.SMEM),
        out_shape=jax.ShapeDtypeStruct((_A, 8), jnp.float32),
    )(gath, labels2, win2, *box2, *boxr)


def kernel(deep, medium, shallow, bboxes, labels):
    bx = bboxes[..., 0]
    by = bboxes[..., 1]
    bw = bboxes[..., 2]
    bh = bboxes[..., 3]
    labels2 = labels.astype(jnp.int32).reshape(_NSC, _BPW)
    box2 = tuple(v.reshape(_NSC, _BPW) for v in (bx, by, bw, bh))
    boxr = tuple(jnp.repeat(v, _NSC // _B, axis=0) for v in (bx, by, bw, bh))

    bases, win = _run_prologue(bx, by, bw, bh)
    gath = _sc_gather(_as_rows128(deep), _as_rows128(medium),
                      _as_rows128(shallow), bases.reshape(_A, 1, _NBOX))
    ep = _run_epilogue(gath.reshape(_A, _NSC, _RPAD // _BPW, _BPW),
                       labels2, win.reshape(_A, _NSC, _BPW), box2, boxr)

    total = jnp.float32(0.0)
    for pred, lt in ((deep, 0), (medium, 1), (shallow, 2)):
        s = _run_dense(pred, bx, by, bw, bh, lt)
        n_pos = jnp.maximum(ep[lt, 5], 1.0)
        l = (_BOX_RATIO * (ep[lt, 0] + ep[lt, 1])
             + _OBJ_RATIO * _BALANCE[lt] * (ep[lt, 2] + s[0] - ep[lt, 3])
             + _CLS_RATIO * ep[lt, 4]) / n_pos
        total = total + l
    return total


# speedup vs baseline: 2.0239x; 2.0239x over previous
import functools

import jax
import jax.numpy as jnp
from jax import lax
from jax.experimental import pallas as pl
from jax.experimental.pallas import tpu as pltpu
from jax.experimental.pallas import tpu_sc as plsc

_NUM_CLASSES = 80
_ATTRS = _NUM_CLASSES + 5
_INPUT_SIZE = 416.0
_ANCHORS = ((10., 13.), (16., 30.), (33., 23.), (30., 61.), (62., 45.),
            (59., 119.), (116., 90.), (156., 198.), (373., 326.))
_MASK = ((6, 7, 8), (3, 4, 5), (0, 1, 2))
_BALANCE = (0.4, 1.0, 4.0)
_BOX_RATIO = 0.05
_OBJ_RATIO = 5.0
_CLS_RATIO = _NUM_CLASSES / 80.0
_EPS = 1e-7
_B = 16
_M = 32
_A = 3
_NBOX = _B * _M
_NSC = 32
_BPW = _NBOX // _NSC
_ROWS = _BPW * _ATTRS
_RPAD = 1408
_HWS = ((13, 13), (26, 26), (52, 52))


def _softplus(x):
    return jnp.maximum(x, 0.0) + jnp.log(1.0 + jnp.exp(-jnp.abs(x)))


def _box_prologue(bxv, byv, bwv, bhv, lt):
    H, W = _HWS[lt]
    stride = _INPUT_SIZE / H
    sa = tuple((aw / stride, ah / stride) for aw, ah in _ANCHORS)
    mids = _MASK[lt]
    Hf, Wf = float(H), float(W)
    gx = bxv * Wf
    gy = byv * Hf
    gw = jnp.maximum(bwv * Wf, 1e-6)
    gh = jnp.maximum(bhv * Hf, 1e-6)
    valid = (bwv > 1e-6) & (bhv > 1e-6)

    def anch_iou(aw, ah):
        inter = jnp.minimum(gw, aw) * jnp.minimum(gh, ah)
        union = gw * gh + aw * ah - inter
        return inter / jnp.maximum(union, 1e-9)

    b_iou = anch_iou(*sa[0])
    b_idx = jnp.zeros(gx.shape, jnp.int32)
    b_aw = jnp.full(gx.shape, sa[0][0], jnp.float32)
    b_ah = jnp.full(gx.shape, sa[0][1], jnp.float32)
    for i in range(1, 9):
        iou_i = anch_iou(*sa[i])
        upd = iou_i > b_iou
        b_iou = jnp.where(upd, iou_i, b_iou)
        b_idx = jnp.where(upd, i, b_idx)
        b_aw = jnp.where(upd, sa[i][0], b_aw)
        b_ah = jnp.where(upd, sa[i][1], b_ah)

    in_layer = ((b_idx == mids[0]) | (b_idx == mids[1]) | (b_idx == mids[2]))
    a_sel = jnp.where(b_idx == mids[0], 0,
                      jnp.where(b_idx == mids[1], 1, 2)).astype(jnp.int32)
    gi = jnp.clip(jnp.floor(gx).astype(jnp.int32), 0, W - 1)
    gj = jnp.clip(jnp.floor(gy).astype(jnp.int32), 0, H - 1)
    ok = valid & in_layer
    return dict(gx=gx, gy=gy, gw=gw, gh=gh, valid=valid, ok=ok,
                a_sel=a_sel, gi=gi, gj=gj, b_aw=b_aw, b_ah=b_ah,
                H=H, W=W)


def _prologue_body(bx_ref, by_ref, bw_ref, bh_ref, base_ref, win_ref):
    bxv = bx_ref[...]
    byv = by_ref[...]
    bwv = bw_ref[...]
    bhv = bh_ref[...]
    bidx = lax.broadcasted_iota(jnp.int32, (_B, _M), 0)
    mi = lax.broadcasted_iota(jnp.int32, (_B, _M, _M), 1)
    mj = lax.broadcasted_iota(jnp.int32, (_B, _M, _M), 2)
    upper = mj > mi
    for lt in range(_A):
        p = _box_prologue(bxv, byv, bwv, bhv, lt)
        H, W = p["H"], p["W"]
        HW = H * W
        ok, a_sel, gi, gj = p["ok"], p["a_sel"], p["gi"], p["gj"]
        key = a_sel * HW + gj * W + gi
        conflict = jnp.any(ok[:, None, :] & (key[:, :, None] == key[:, None, :])
                           & upper, axis=2)
        win = ok & jnp.logical_not(conflict)
        base = ((gj * W + gi) * _B + bidx) * (_A * _ATTRS) + a_sel * _ATTRS
        base_ref[lt] = jnp.where(win, base, 0)
        win_ref[lt] = win.astype(jnp.int32)


def _run_prologue(bx, by, bw, bh):
    spec = pl.BlockSpec((_B, _M), lambda _: (0, 0))
    out = pl.pallas_call(
        _prologue_body,
        grid=(1,),
        in_specs=[spec, spec, spec, spec],
        out_specs=(pl.BlockSpec((_A, _B, _M), lambda _: (0, 0, 0)),
                   pl.BlockSpec((_A, _B, _M), lambda _: (0, 0, 0))),
        out_shape=(jax.ShapeDtypeStruct((_A, _B, _M), jnp.int32),
                   jax.ShapeDtypeStruct((_A, _B, _M), jnp.int32)),
    )(bx, by, bw, bh)
    return out


_NCH = -(-_ROWS // 128)


_NBUF = 4


def _sc_body(xd_ref, xm_ref, xs_ref, bases_ref, out_ref,
             row_v, out_v, rows_v, base_v, sem):
    nc = 2
    wid = lax.axis_index("s") * nc + lax.axis_index("c")
    tables = (xd_ref, xm_ref, xs_ref)
    lane16 = lax.broadcasted_iota(jnp.int32, (_BPW,), 0)
    for l in range(_A):
        pltpu.sync_copy(bases_ref.at[l, 0, pl.ds(wid * _BPW, _BPW)], base_v)
        bvec = base_v[...]
        r0v = jnp.right_shift(bvec, 7)
        lane0v = jnp.bitwise_and(bvec, 127)
        row_v[pl.ds(0, _BPW)] = r0v
        row_v[pl.ds(_BPW, _BPW)] = r0v + 1
        cp = pltpu.async_copy(tables[l].at[row_v], rows_v, sem)
        cp.wait()

        def extract(c, carry):
            lane = lane0v + c
            rr = lane16 + _BPW * jnp.right_shift(lane, 7)
            vals = plsc.load_gather(rows_v, [rr, jnp.bitwise_and(lane, 127)])
            out_v[pl.ds(c * _BPW, _BPW)] = vals
            return carry

        lax.fori_loop(0, _ATTRS, extract, 0)
        pltpu.sync_copy(out_v, out_ref.at[l, 0, pl.ds(wid * _RPAD, _RPAD)])


def _sc_gather(xd, xm, xs, bases):
    mesh = plsc.VectorSubcoreMesh(core_axis_name="c", subcore_axis_name="s")
    f = pl.kernel(
        _sc_body,
        out_type=jax.ShapeDtypeStruct((_A, 1, _NSC * _RPAD), jnp.float32),
        mesh=mesh,
        compiler_params=pltpu.CompilerParams(needs_layout_passes=False),
        scratch_types=[
            pltpu.VMEM((2 * _BPW,), jnp.int32),
            pltpu.VMEM((_RPAD,), jnp.float32),
            pltpu.VMEM((2 * _BPW, 128), jnp.float32),
            pltpu.VMEM((_BPW,), jnp.int32),
            pltpu.SemaphoreType.DMA,
        ],
    )
    return f(xd, xm, xs, bases)


def _pad_copy_body(x_ref, o_ref):
    o_ref[...] = x_ref[...]


def _as_rows128(pred):
    f = jnp.transpose(pred, (2, 3, 0, 1)).reshape(-1)
    n = f.shape[0]
    npad = -(-n // 128) * 128
    if npad == n:
        return f.reshape(-1, 128)
    blk = 128 * 1024
    grid = -(-npad // blk)
    out = pl.pallas_call(
        _pad_copy_body,
        grid=(grid,),
        in_specs=[pl.BlockSpec((blk,), lambda i: (i,))],
        out_specs=pl.BlockSpec((blk,), lambda i: (i,)),
        out_shape=jax.ShapeDtypeStruct((npad,), jnp.float32),
    )(f)
    return out.reshape(-1, 128)


def _dense_body(lt, bx_ref, by_ref, bw_ref, bh_ref,
                x0_ref, x1_ref, x2_ref, out_ref):
    H, W = _HWS[lt]
    HW = H * W
    stride = _INPUT_SIZE / H
    sa = tuple((aw / stride, ah / stride) for aw, ah in _ANCHORS)
    mids = _MASK[lt]
    Hf, Wf = float(H), float(W)
    xrefs = (x0_ref, x1_ref, x2_ref)

    p = _box_prologue(bx_ref[...], by_ref[...], bw_ref[...], bh_ref[...], lt)
    gx, gy, gw, gh, valid = p["gx"], p["gy"], p["gw"], p["gh"], p["valid"]
    g1x = gx - gw / 2
    g2x = gx + gw / 2
    g1y = gy - gh / 2
    g2y = gy + gh / 2
    area_g = (g2x - g1x) * (g2y - g1y)

    q = lax.broadcasted_iota(jnp.int32, (_B, HW), 1)
    jj = q // W
    ii = q - jj * W
    iif = ii.astype(jnp.float32)
    jjf = jj.astype(jnp.float32)

    s4 = 0.0
    for a in range(_A):
        X = xrefs[a]
        xp = X[:, 0, 0, :]
        yp = X[:, 0, 1, :]
        pw = X[:, 0, 2, :]
        ph = X[:, 0, 3, :]
        xc = X[:, 0, 4, :]
        law, lah = sa[mids[a]]
        bxg = jax.nn.sigmoid(xp) + iif
        byg = jax.nn.sigmoid(yp) + jjf
        bwg = jnp.exp(pw) * law
        bhg = jnp.exp(ph) * lah
        p1x = bxg - bwg / 2
        p2x = bxg + bwg / 2
        p1y = byg - bhg / 2
        p2y = byg + bhg / 2
        area_p = (p2x - p1x) * (p2y - p1y)

        ign = jnp.zeros((_B, HW), jnp.bool_)
        for m in range(_M):
            sl = lambda arr: arr[:, m:m + 1]
            iw = jnp.maximum(jnp.minimum(p2x, sl(g2x)) - jnp.maximum(p1x, sl(g1x)), 0.0)
            ih = jnp.maximum(jnp.minimum(p2y, sl(g2y)) - jnp.maximum(p1y, sl(g1y)), 0.0)
            inter = iw * ih
            den = jnp.maximum(area_p + sl(area_g) - inter, 1e-9)
            ign = ign | (sl(valid) & (inter + inter > den))

        s4 = s4 + jnp.sum(jnp.logical_not(ign).astype(jnp.float32) * _softplus(xc))

    out_ref[0] = s4


def _run_dense(pred, bx, by, bw, bh, lt):
    B, C, H, W = pred.shape
    HW = H * W
    x = jnp.concatenate([pred[:, a * _ATTRS:a * _ATTRS + 5] for a in range(_A)],
                        axis=1).reshape(B, _A, 5, HW)
    vec_spec = pl.BlockSpec((_B, _M), lambda _: (0, 0))

    def xspec(a):
        return pl.BlockSpec((B, 1, 5, HW), lambda _, a=a: (0, a, 0, 0))

    return pl.pallas_call(
        functools.partial(_dense_body, lt),
        grid=(1,),
        in_specs=[vec_spec, vec_spec, vec_spec, vec_spec,
                  xspec(0), xspec(1), xspec(2)],
        out_specs=pl.BlockSpec(memory_space=pltpu.SMEM),
        out_shape=jax.ShapeDtypeStruct((8,), jnp.float32),
    )(bx, by, bw, bh, x, x, x)


def _epilogue_body(g_ref, lab_ref, win_ref, bx_ref, by_ref, bw_ref, bh_ref,
                   bxr_ref, byr_ref, bwr_ref, bhr_ref, out_ref):
    lab = lab_ref[...]
    cid = lax.broadcasted_iota(jnp.int32, (_NSC, _NUM_CLASSES, _BPW), 1)
    tt = (lab[:, None, :] == cid).astype(jnp.float32)
    bxv, byv, bwv, bhv = bx_ref[...], by_ref[...], bw_ref[...], bh_ref[...]
    bxr, byr, bwr, bhr = bxr_ref[...], byr_ref[...], bwr_ref[...], bhr_ref[...]
    for l in range(_A):
        H, W = _HWS[l]
        stride = _INPUT_SIZE / H
        sa = tuple((aw / stride, ah / stride) for aw, ah in _ANCHORS)
        mids = _MASK[l]
        Hf, Wf = float(H), float(W)

        p = _box_prologue(bxv, byv, bwv, bhv, l)
        gx, gy, gw, gh = p["gx"], p["gy"], p["gw"], p["gh"]
        a_sel, gi, gj = p["a_sel"], p["gi"], p["gj"]
        b_aw, b_ah = p["b_aw"], p["b_ah"]
        gif = gi.astype(jnp.float32)
        gjf = gj.astype(jnp.float32)
        tx = gx - gif
        ty = gy - gjf
        tw = jnp.log(gw / b_aw)
        th = jnp.log(gh / b_ah)
        scl = gw * gh / (Hf * Wf)

        wf = (win_ref[l] > 0).astype(jnp.float32)
        X = g_ref[l]
        xp = X[:, 0, :]
        yp = X[:, 1, :]
        pw = X[:, 2, :]
        ph = X[:, 3, :]
        xc = X[:, 4, :]
        cls = X[:, 5:5 + _NUM_CLASSES, :]

        bsc = 2.0 - scl
        s1 = jnp.sum(wf * bsc * (_softplus(xp) - tx * xp + _softplus(yp) - ty * yp))
        s2 = jnp.sum(wf * bsc * 0.5 * ((pw - tw) ** 2 + (ph - th) ** 2))
        spc = _softplus(xc)
        s3 = jnp.sum(wf * (spc - xc))

        law = jnp.where(a_sel == 0, sa[mids[0]][0],
                        jnp.where(a_sel == 1, sa[mids[1]][0], sa[mids[2]][0]))
        lah = jnp.where(a_sel == 0, sa[mids[0]][1],
                        jnp.where(a_sel == 1, sa[mids[1]][1], sa[mids[2]][1]))
        bxg = jax.nn.sigmoid(xp) + gif
        byg = jax.nn.sigmoid(yp) + gjf
        bwg = jnp.exp(pw) * law
        bhg = jnp.exp(ph) * lah
        p1x = bxg - bwg / 2
        p2x = bxg + bwg / 2
        p1y = byg - bhg / 2
        p2y = byg + bhg / 2
        area_p = (p2x - p1x) * (p2y - p1y)

        gxr = bxr * Wf
        gyr = byr * Hf
        gwr = jnp.maximum(bwr * Wf, 1e-6)
        ghr = jnp.maximum(bhr * Hf, 1e-6)
        validr = (bwr > 1e-6) & (bhr > 1e-6)
        g1xr = gxr - gwr / 2
        g2xr = gxr + gwr / 2
        g1yr = gyr - ghr / 2
        g2yr = gyr + ghr / 2
        area_gr = (g2xr - g1xr) * (g2yr - g1yr)

        ign = jnp.zeros((_NSC, _BPW), jnp.bool_)
        for m in range(_M):
            sl = lambda arr: arr[:, m:m + 1]
            iw = jnp.maximum(jnp.minimum(p2x, sl(g2xr)) - jnp.maximum(p1x, sl(g1xr)), 0.0)
            ih = jnp.maximum(jnp.minimum(p2y, sl(g2yr)) - jnp.maximum(p1y, sl(g1yr)), 0.0)
            inter = iw * ih
            den = jnp.maximum(area_p + sl(area_gr) - inter, 1e-9)
            ign = ign | (sl(validr) & (inter + inter > den))

        s4c = jnp.sum(wf * jnp.logical_not(ign).astype(jnp.float32) * spc)
        s5 = jnp.sum(wf[:, None, :] * (_softplus(cls) - tt * cls))
        npos = jnp.sum(wf)

        out_ref[l, 0] = s1
        out_ref[l, 1] = s2
        out_ref[l, 2] = s3
        out_ref[l, 3] = s4c
        out_ref[l, 4] = s5
        out_ref[l, 5] = npos


def _run_epilogue(gath, labels2, win2, box2, boxr):
    s2d = pl.BlockSpec((_NSC, _BPW), lambda _: (0, 0))
    srep = pl.BlockSpec((_NSC, _M), lambda _: (0, 0))
    return pl.pallas_call(
        _epilogue_body,
        grid=(1,),
        in_specs=[pl.BlockSpec((_A, _NSC, _RPAD // _BPW, _BPW),
                               lambda _: (0, 0, 0, 0)),
                  s2d,
                  pl.BlockSpec((_A, _NSC, _BPW), lambda _: (0, 0, 0)),
                  s2d, s2d, s2d, s2d,
                  srep, srep, srep, srep],
        out_specs=pl.BlockSpec(memory_space=pltpu.SMEM),
        out_shape=jax.ShapeDtypeStruct((_A, 8), jnp.float32),
    )(gath, labels2, win2, *box2, *boxr)


def kernel(deep, medium, shallow, bboxes, labels):
    bx = bboxes[..., 0]
    by = bboxes[..., 1]
    bw = bboxes[..., 2]
    bh = bboxes[..., 3]
    labels2 = labels.astype(jnp.int32).reshape(_NSC, _BPW)
    box2 = tuple(v.reshape(_NSC, _BPW) for v in (bx, by, bw, bh))
    boxr = tuple(jnp.repeat(v, _NSC // _B, axis=0) for v in (bx, by, bw, bh))

    bases, win = _run_prologue(bx, by, bw, bh)
    gath = _sc_gather(_as_rows128(deep), _as_rows128(medium),
                      _as_rows128(shallow), bases.reshape(_A, 1, _NBOX))
    ep = _run_epilogue(gath.reshape(_A, _NSC, _RPAD // _BPW, _BPW),
                       labels2, win.reshape(_A, _NSC, _BPW), box2, boxr)

    total = jnp.float32(0.0)
    for pred, lt in ((deep, 0), (medium, 1), (shallow, 2)):
        s = _run_dense(pred, bx, by, bw, bh, lt)
        n_pos = jnp.maximum(ep[lt, 5], 1.0)
        l = (_BOX_RATIO * (ep[lt, 0] + ep[lt, 1])
             + _OBJ_RATIO * _BALANCE[lt] * (ep[lt, 2] + s[0] - ep[lt, 3])
             + _CLS_RATIO * ep[lt, 4]) / n_pos
        total = total + l
    return total


# speedup vs baseline: 2.0377x; 1.0068x over previous
import functools

import jax
import jax.numpy as jnp
from jax import lax
from jax.experimental import pallas as pl
from jax.experimental.pallas import tpu as pltpu
from jax.experimental.pallas import tpu_sc as plsc

_NUM_CLASSES = 80
_ATTRS = _NUM_CLASSES + 5
_INPUT_SIZE = 416.0
_ANCHORS = ((10., 13.), (16., 30.), (33., 23.), (30., 61.), (62., 45.),
            (59., 119.), (116., 90.), (156., 198.), (373., 326.))
_MASK = ((6, 7, 8), (3, 4, 5), (0, 1, 2))
_BALANCE = (0.4, 1.0, 4.0)
_BOX_RATIO = 0.05
_OBJ_RATIO = 5.0
_CLS_RATIO = _NUM_CLASSES / 80.0
_EPS = 1e-7
_B = 16
_M = 32
_A = 3
_NBOX = _B * _M
_NSC = 32
_BPW = _NBOX // _NSC
_ROWS = _BPW * _ATTRS
_RPAD = 1408
_HWS = ((13, 13), (26, 26), (52, 52))


def _softplus(x):
    return jnp.maximum(x, 0.0) + jnp.log(1.0 + jnp.exp(-jnp.abs(x)))


def _box_prologue(bxv, byv, bwv, bhv, lt):
    H, W = _HWS[lt]
    stride = _INPUT_SIZE / H
    sa = tuple((aw / stride, ah / stride) for aw, ah in _ANCHORS)
    mids = _MASK[lt]
    Hf, Wf = float(H), float(W)
    gx = bxv * Wf
    gy = byv * Hf
    gw = jnp.maximum(bwv * Wf, 1e-6)
    gh = jnp.maximum(bhv * Hf, 1e-6)
    valid = (bwv > 1e-6) & (bhv > 1e-6)

    def anch_iou(aw, ah):
        inter = jnp.minimum(gw, aw) * jnp.minimum(gh, ah)
        union = gw * gh + aw * ah - inter
        return inter / jnp.maximum(union, 1e-9)

    b_iou = anch_iou(*sa[0])
    b_idx = jnp.zeros(gx.shape, jnp.int32)
    b_aw = jnp.full(gx.shape, sa[0][0], jnp.float32)
    b_ah = jnp.full(gx.shape, sa[0][1], jnp.float32)
    for i in range(1, 9):
        iou_i = anch_iou(*sa[i])
        upd = iou_i > b_iou
        b_iou = jnp.where(upd, iou_i, b_iou)
        b_idx = jnp.where(upd, i, b_idx)
        b_aw = jnp.where(upd, sa[i][0], b_aw)
        b_ah = jnp.where(upd, sa[i][1], b_ah)

    in_layer = ((b_idx == mids[0]) | (b_idx == mids[1]) | (b_idx == mids[2]))
    a_sel = jnp.where(b_idx == mids[0], 0,
                      jnp.where(b_idx == mids[1], 1, 2)).astype(jnp.int32)
    gi = jnp.clip(jnp.floor(gx).astype(jnp.int32), 0, W - 1)
    gj = jnp.clip(jnp.floor(gy).astype(jnp.int32), 0, H - 1)
    ok = valid & in_layer
    return dict(gx=gx, gy=gy, gw=gw, gh=gh, valid=valid, ok=ok,
                a_sel=a_sel, gi=gi, gj=gj, b_aw=b_aw, b_ah=b_ah,
                H=H, W=W)


def _prologue_body(bx_ref, by_ref, bw_ref, bh_ref, base_ref, win_ref):
    bxv = bx_ref[...]
    byv = by_ref[...]
    bwv = bw_ref[...]
    bhv = bh_ref[...]
    bidx = lax.broadcasted_iota(jnp.int32, (_B, _M), 0)
    mi = lax.broadcasted_iota(jnp.int32, (_B, _M, _M), 1)
    mj = lax.broadcasted_iota(jnp.int32, (_B, _M, _M), 2)
    upper = mj > mi
    for lt in range(_A):
        p = _box_prologue(bxv, byv, bwv, bhv, lt)
        H, W = p["H"], p["W"]
        HW = H * W
        ok, a_sel, gi, gj = p["ok"], p["a_sel"], p["gi"], p["gj"]
        key = a_sel * HW + gj * W + gi
        conflict = jnp.any(ok[:, None, :] & (key[:, :, None] == key[:, None, :])
                           & upper, axis=2)
        win = ok & jnp.logical_not(conflict)
        base = ((gj * W + gi) * _B + bidx) * (_A * _ATTRS) + a_sel * _ATTRS
        base_ref[lt] = jnp.where(win, base, 0)
        win_ref[lt] = win.astype(jnp.int32)


def _run_prologue(bx, by, bw, bh):
    spec = pl.BlockSpec((_B, _M), lambda _: (0, 0))
    out = pl.pallas_call(
        _prologue_body,
        grid=(1,),
        in_specs=[spec, spec, spec, spec],
        out_specs=(pl.BlockSpec((_A, _B, _M), lambda _: (0, 0, 0)),
                   pl.BlockSpec((_A, _B, _M), lambda _: (0, 0, 0))),
        out_shape=(jax.ShapeDtypeStruct((_A, _B, _M), jnp.int32),
                   jax.ShapeDtypeStruct((_A, _B, _M), jnp.int32)),
    )(bx, by, bw, bh)
    return out


_NCH = -(-_ROWS // 128)


_NBUF = 4


def _sc_body(xd_ref, xm_ref, xs_ref, bases_ref, out_ref,
             row_v, out_v, rows_v, base_v, sem):
    nc = 2
    wid = lax.axis_index("s") * nc + lax.axis_index("c")
    tables = (xd_ref, xm_ref, xs_ref)
    lane16 = lax.broadcasted_iota(jnp.int32, (_BPW,), 0)
    for l in range(_A):
        pltpu.sync_copy(bases_ref.at[l, 0, pl.ds(wid * _BPW, _BPW)], base_v)
        bvec = base_v[...]
        r0v = jnp.right_shift(bvec, 7)
        lane0v = jnp.bitwise_and(bvec, 127)
        row_v[pl.ds(0, _BPW)] = r0v
        row_v[pl.ds(_BPW, _BPW)] = r0v + 1
        cp = pltpu.async_copy(tables[l].at[row_v], rows_v, sem)
        cp.wait()

        def extract(c, carry):
            lane = lane0v + c
            rr = lane16 + _BPW * jnp.right_shift(lane, 7)
            vals = plsc.load_gather(rows_v, [rr, jnp.bitwise_and(lane, 127)])
            out_v[pl.ds(c * _BPW, _BPW)] = vals
            return carry

        lax.fori_loop(0, _ATTRS, extract, 0)
        pltpu.sync_copy(out_v, out_ref.at[l, 0, pl.ds(wid * _RPAD, _RPAD)])


def _sc_gather(xd, xm, xs, bases):
    mesh = plsc.VectorSubcoreMesh(core_axis_name="c", subcore_axis_name="s")
    f = pl.kernel(
        _sc_body,
        out_type=jax.ShapeDtypeStruct((_A, 1, _NSC * _RPAD), jnp.float32),
        mesh=mesh,
        compiler_params=pltpu.CompilerParams(needs_layout_passes=False),
        scratch_types=[
            pltpu.VMEM((2 * _BPW,), jnp.int32),
            pltpu.VMEM((_RPAD,), jnp.float32),
            pltpu.VMEM((2 * _BPW, 128), jnp.float32),
            pltpu.VMEM((_BPW,), jnp.int32),
            pltpu.SemaphoreType.DMA,
        ],
    )
    return f(xd, xm, xs, bases)


def _pad_copy_body(x_ref, o_ref):
    o_ref[...] = x_ref[...]


def _as_rows128(pred):
    f = jnp.transpose(pred, (2, 3, 0, 1)).reshape(-1)
    n = f.shape[0]
    npad = -(-n // 128) * 128
    if npad == n:
        return f.reshape(-1, 128)
    blk = 128 * 1024
    grid = -(-npad // blk)
    out = pl.pallas_call(
        _pad_copy_body,
        grid=(grid,),
        in_specs=[pl.BlockSpec((blk,), lambda i: (i,))],
        out_specs=pl.BlockSpec((blk,), lambda i: (i,)),
        out_shape=jax.ShapeDtypeStruct((npad,), jnp.float32),
    )(f)
    return out.reshape(-1, 128)


def _dense_body(lt, bx_ref, by_ref, bw_ref, bh_ref, x0_ref, out_ref):
    H, W = _HWS[lt]
    HW = H * W
    stride = _INPUT_SIZE / H
    sa = tuple((aw / stride, ah / stride) for aw, ah in _ANCHORS)
    mids = _MASK[lt]
    Hf, Wf = float(H), float(W)
    xrefs = (x0_ref,)

    a = pl.program_id(0)
    p = _box_prologue(bx_ref[...], by_ref[...], bw_ref[...], bh_ref[...], lt)
    gx, gy, gw, gh, valid = p["gx"], p["gy"], p["gw"], p["gh"], p["valid"]
    g1x = gx - gw / 2
    g2x = gx + gw / 2
    g1y = gy - gh / 2
    g2y = gy + gh / 2
    area_g = (g2x - g1x) * (g2y - g1y)

    q = lax.broadcasted_iota(jnp.int32, (_B, HW), 1)
    jj = q // W
    ii = q - jj * W
    iif = ii.astype(jnp.float32)
    jjf = jj.astype(jnp.float32)

    X = xrefs[0]
    xp = X[:, 0, 0, :]
    yp = X[:, 0, 1, :]
    pw = X[:, 0, 2, :]
    ph = X[:, 0, 3, :]
    xc = X[:, 0, 4, :]
    la0, la1, la2 = sa[mids[0]], sa[mids[1]], sa[mids[2]]
    law = jnp.where(a == 0, la0[0], jnp.where(a == 1, la1[0], la2[0]))
    lah = jnp.where(a == 0, la0[1], jnp.where(a == 1, la1[1], la2[1]))
    bxg = jax.nn.sigmoid(xp) + iif
    byg = jax.nn.sigmoid(yp) + jjf
    bwg = jnp.exp(pw) * law
    bhg = jnp.exp(ph) * lah
    p1x = bxg - bwg / 2
    p2x = bxg + bwg / 2
    p1y = byg - bhg / 2
    p2y = byg + bhg / 2
    area_p = (p2x - p1x) * (p2y - p1y)

    ign = jnp.zeros((_B, HW), jnp.bool_)
    for m in range(_M):
        sl = lambda arr: arr[:, m:m + 1]
        iw = jnp.maximum(jnp.minimum(p2x, sl(g2x)) - jnp.maximum(p1x, sl(g1x)), 0.0)
        ih = jnp.maximum(jnp.minimum(p2y, sl(g2y)) - jnp.maximum(p1y, sl(g1y)), 0.0)
        inter = iw * ih
        den = jnp.maximum(area_p + sl(area_g) - inter, 1e-9)
        ign = ign | (sl(valid) & (inter + inter > den))

    s4 = jnp.sum(jnp.logical_not(ign).astype(jnp.float32) * _softplus(xc))

    @pl.when(a == 0)
    def _init():
        out_ref[0] = s4

    @pl.when(a != 0)
    def _acc():
        out_ref[0] = out_ref[0] + s4


def _run_dense(pred, bx, by, bw, bh, lt):
    B, C, H, W = pred.shape
    HW = H * W
    x = jnp.concatenate([pred[:, a * _ATTRS:a * _ATTRS + 5] for a in range(_A)],
                        axis=1).reshape(B, _A, 5, HW)
    vec_spec = pl.BlockSpec((_B, _M), lambda _: (0, 0))
    xspec = pl.BlockSpec((B, 1, 5, HW), lambda a: (0, a, 0, 0))

    return pl.pallas_call(
        functools.partial(_dense_body, lt),
        grid=(_A,),
        in_specs=[vec_spec, vec_spec, vec_spec, vec_spec, xspec],
        out_specs=pl.BlockSpec(memory_space=pltpu.SMEM),
        out_shape=jax.ShapeDtypeStruct((8,), jnp.float32),
    )(bx, by, bw, bh, x)


def _epilogue_body(g_ref, lab_ref, win_ref, bx_ref, by_ref, bw_ref, bh_ref,
                   bxr_ref, byr_ref, bwr_ref, bhr_ref, out_ref):
    lab = lab_ref[...]
    cid = lax.broadcasted_iota(jnp.int32, (_NSC, _NUM_CLASSES, _BPW), 1)
    tt = (lab[:, None, :] == cid).astype(jnp.float32)
    bxv, byv, bwv, bhv = bx_ref[...], by_ref[...], bw_ref[...], bh_ref[...]
    bxr, byr, bwr, bhr = bxr_ref[...], byr_ref[...], bwr_ref[...], bhr_ref[...]
    for l in range(_A):
        H, W = _HWS[l]
        stride = _INPUT_SIZE / H
        sa = tuple((aw / stride, ah / stride) for aw, ah in _ANCHORS)
        mids = _MASK[l]
        Hf, Wf = float(H), float(W)

        p = _box_prologue(bxv, byv, bwv, bhv, l)
        gx, gy, gw, gh = p["gx"], p["gy"], p["gw"], p["gh"]
        a_sel, gi, gj = p["a_sel"], p["gi"], p["gj"]
        b_aw, b_ah = p["b_aw"], p["b_ah"]
        gif = gi.astype(jnp.float32)
        gjf = gj.astype(jnp.float32)
        tx = gx - gif
        ty = gy - gjf
        tw = jnp.log(gw / b_aw)
        th = jnp.log(gh / b_ah)
        scl = gw * gh / (Hf * Wf)

        wf = (win_ref[l] > 0).astype(jnp.float32)
        X = g_ref[l]
        xp = X[:, 0, :]
        yp = X[:, 1, :]
        pw = X[:, 2, :]
        ph = X[:, 3, :]
        xc = X[:, 4, :]
        cls = X[:, 5:5 + _NUM_CLASSES, :]

        bsc = 2.0 - scl
        s1 = jnp.sum(wf * bsc * (_softplus(xp) - tx * xp + _softplus(yp) - ty * yp))
        s2 = jnp.sum(wf * bsc * 0.5 * ((pw - tw) ** 2 + (ph - th) ** 2))
        spc = _softplus(xc)
        s3 = jnp.sum(wf * (spc - xc))

        law = jnp.where(a_sel == 0, sa[mids[0]][0],
                        jnp.where(a_sel == 1, sa[mids[1]][0], sa[mids[2]][0]))
        lah = jnp.where(a_sel == 0, sa[mids[0]][1],
                        jnp.where(a_sel == 1, sa[mids[1]][1], sa[mids[2]][1]))
        bxg = jax.nn.sigmoid(xp) + gif
        byg = jax.nn.sigmoid(yp) + gjf
        bwg = jnp.exp(pw) * law
        bhg = jnp.exp(ph) * lah
        p1x = bxg - bwg / 2
        p2x = bxg + bwg / 2
        p1y = byg - bhg / 2
        p2y = byg + bhg / 2
        area_p = (p2x - p1x) * (p2y - p1y)

        gxr = bxr * Wf
        gyr = byr * Hf
        gwr = jnp.maximum(bwr * Wf, 1e-6)
        ghr = jnp.maximum(bhr * Hf, 1e-6)
        validr = (bwr > 1e-6) & (bhr > 1e-6)
        g1xr = gxr - gwr / 2
        g2xr = gxr + gwr / 2
        g1yr = gyr - ghr / 2
        g2yr = gyr + ghr / 2
        area_gr = (g2xr - g1xr) * (g2yr - g1yr)

        ign = jnp.zeros((_NSC, _BPW), jnp.bool_)
        for m in range(_M):
            sl = lambda arr: arr[:, m:m + 1]
            iw = jnp.maximum(jnp.minimum(p2x, sl(g2xr)) - jnp.maximum(p1x, sl(g1xr)), 0.0)
            ih = jnp.maximum(jnp.minimum(p2y, sl(g2yr)) - jnp.maximum(p1y, sl(g1yr)), 0.0)
            inter = iw * ih
            den = jnp.maximum(area_p + sl(area_gr) - inter, 1e-9)
            ign = ign | (sl(validr) & (inter + inter > den))

        s4c = jnp.sum(wf * jnp.logical_not(ign).astype(jnp.float32) * spc)
        s5 = jnp.sum(wf[:, None, :] * (_softplus(cls) - tt * cls))
        npos = jnp.sum(wf)

        out_ref[l, 0] = s1
        out_ref[l, 1] = s2
        out_ref[l, 2] = s3
        out_ref[l, 3] = s4c
        out_ref[l, 4] = s5
        out_ref[l, 5] = npos


def _run_epilogue(gath, labels2, win2, box2, boxr):
    s2d = pl.BlockSpec((_NSC, _BPW), lambda _: (0, 0))
    srep = pl.BlockSpec((_NSC, _M), lambda _: (0, 0))
    return pl.pallas_call(
        _epilogue_body,
        grid=(1,),
        in_specs=[pl.BlockSpec((_A, _NSC, _RPAD // _BPW, _BPW),
                               lambda _: (0, 0, 0, 0)),
                  s2d,
                  pl.BlockSpec((_A, _NSC, _BPW), lambda _: (0, 0, 0)),
                  s2d, s2d, s2d, s2d,
                  srep, srep, srep, srep],
        out_specs=pl.BlockSpec(memory_space=pltpu.SMEM),
        out_shape=jax.ShapeDtypeStruct((_A, 8), jnp.float32),
    )(gath, labels2, win2, *box2, *boxr)


def kernel(deep, medium, shallow, bboxes, labels):
    bx = bboxes[..., 0]
    by = bboxes[..., 1]
    bw = bboxes[..., 2]
    bh = bboxes[..., 3]
    labels2 = labels.astype(jnp.int32).reshape(_NSC, _BPW)
    box2 = tuple(v.reshape(_NSC, _BPW) for v in (bx, by, bw, bh))
    boxr = tuple(jnp.repeat(v, _NSC // _B, axis=0) for v in (bx, by, bw, bh))

    bases, win = _run_prologue(bx, by, bw, bh)
    gath = _sc_gather(_as_rows128(deep), _as_rows128(medium),
                      _as_rows128(shallow), bases.reshape(_A, 1, _NBOX))
    ep = _run_epilogue(gath.reshape(_A, _NSC, _RPAD // _BPW, _BPW),
                       labels2, win.reshape(_A, _NSC, _BPW), box2, boxr)

    total = jnp.float32(0.0)
    for pred, lt in ((deep, 0), (medium, 1), (shallow, 2)):
        s = _run_dense(pred, bx, by, bw, bh, lt)
        n_pos = jnp.maximum(ep[lt, 5], 1.0)
        l = (_BOX_RATIO * (ep[lt, 0] + ep[lt, 1])
             + _OBJ_RATIO * _BALANCE[lt] * (ep[lt, 2] + s[0] - ep[lt, 3])
             + _CLS_RATIO * ep[lt, 4]) / n_pos
        total = total + l
    return total
